# baseline (speedup 1.0000x reference)
import functools

import jax
import jax.numpy as jnp
from jax import lax
from jax.experimental import pallas as pl
from jax.experimental.pallas import tpu as pltpu
from jax.experimental.pallas import tpu_sc as plsc

D_MODEL = 2048
MAX_LEN = 2048
VOCAB = 80
BATCH = 4

ROWS = BATCH * MAX_LEN

_INFO = plsc.get_sparse_core_info()
NC, NS, L = _INFO.num_cores, _INFO.num_subcores, _INFO.num_lanes
NW = NC * NS
PPW = MAX_LEN // NW
KP = 4
KR = BATCH * KP
NSTEP = PPW // KP
NBUF = 2


def _pe_table():
    even_i = jnp.arange(0, D_MODEL, 2, dtype=jnp.float32)
    denominator = jnp.power(10000.0, even_i / D_MODEL)
    position = jnp.arange(MAX_LEN, dtype=jnp.float32).reshape(MAX_LEN, 1)
    even_pe = jnp.sin(position / denominator)
    odd_pe = jnp.cos(position / denominator)
    return jnp.stack([even_pe, odd_pe], axis=2).reshape(MAX_LEN, D_MODEL)


def _sc_embed(tokens, table, pe):
    mesh = plsc.VectorSubcoreMesh(core_axis_name="c", subcore_axis_name="s")

    @functools.partial(
        pl.kernel,
        mesh=mesh,
        out_type=jax.ShapeDtypeStruct((ROWS, D_MODEL), jnp.float32),
        scratch_types=[
            pltpu.VMEM_SHARED((NS, KR, D_MODEL), jnp.float32),
            pltpu.VMEM((PPW * BATCH,), jnp.int32),
            pltpu.VMEM((KR, D_MODEL), jnp.float32),
            pltpu.VMEM((KR, D_MODEL), jnp.float32),
            pltpu.VMEM((KP, D_MODEL), jnp.float32),
            pltpu.VMEM((KP, D_MODEL), jnp.float32),
            pltpu.SemaphoreType.DMA,
            pltpu.SemaphoreType.DMA,
            pltpu.SemaphoreType.DMA,
            pltpu.SemaphoreType.DMA,
            pltpu.SemaphoreType.DMA,
            pltpu.SemaphoreType.DMA,
            pltpu.SemaphoreType.DMA,
            pltpu.SemaphoreType.DMA,
        ],
    )
    def k(tok_hbm, table_hbm, pe_hbm, out_hbm, shared, idx2,
          g0, g1, p0b, p1b,
          sg0, sg1, sp0, sp1, so0, so1, sx0, sx1):
        wid = lax.axis_index("s") * NC + lax.axis_index("c")
        sid = lax.axis_index("s")
        pos0 = wid * PPW
        SX = [sx0, sx1]

        G = [g0, g1]
        P = [p0b, p1b]
        SG = [sg0, sg1]
        SP = [sp0, sp1]
        SO = [so0, so1]

        pltpu.sync_copy(tok_hbm.at[pl.ds(wid * PPW * BATCH, PPW * BATCH)],
                        idx2)

        def start_in(s):
            kbuf = s % NBUF
            pltpu.async_copy(
                table_hbm.at[idx2.at[pl.ds(s * KR, KR)]], G[kbuf], SG[kbuf])
            pltpu.async_copy(pe_hbm.at[pl.ds(pos0 + s * KP, KP)],
                             P[kbuf], SP[kbuf])

        def wait_in(s):
            kbuf = s % NBUF
            pltpu.make_async_copy(
                table_hbm.at[idx2.at[pl.ds(s * KR, KR)]],
                G[kbuf], SG[kbuf]).wait()
            pltpu.make_async_copy(
                pe_hbm.at[pl.ds(pos0 + s * KP, KP)], P[kbuf], SP[kbuf]).wait()

        def add(s):
            kbuf = s % NBUF
            g, pbuf = G[kbuf], P[kbuf]

            def body(i, acc):
                for p in range(KP):
                    pv = pbuf[p, pl.ds(i * L, L)]
                    for b in range(BATCH):
                        j = b * KP + p
                        g[j, pl.ds(i * L, L)] = g[j, pl.ds(i * L, L)] + pv
                return acc
            lax.fori_loop(0, D_MODEL // L, body, 0)

        def slot(s):
            return shared.at[sid]

        def start_xbar(s):
            kbuf = s % NBUF
            pltpu.async_copy(G[kbuf], slot(s), SX[kbuf])

        def wait_xbar(s):
            kbuf = s % NBUF
            pltpu.make_async_copy(G[kbuf], slot(s), SX[kbuf]).wait()

        def start_out(s):
            so = SO[s % 2]
            for b in range(BATCH):
                pltpu.async_copy(
                    slot(s).at[pl.ds(b * KP, KP)],
                    out_hbm.at[pl.ds(b * MAX_LEN + pos0 + s * KP, KP)],
                    so)

        def wait_out(s):
            so = SO[s % 2]
            for b in range(BATCH):
                pltpu.make_async_copy(
                    slot(s).at[pl.ds(b * KP, KP)],
                    out_hbm.at[pl.ds(b * MAX_LEN + pos0 + s * KP, KP)],
                    so).wait()

        start_in(0)
        start_in(1)
        for s in range(NSTEP):
            wait_in(s)
            add(s)
            if s >= 1:
                wait_out(s - 1)
            start_xbar(s)
            wait_xbar(s)
            if s + 2 < NSTEP:
                start_in(s + 2)
            start_out(s)
        wait_out(NSTEP - 1)

    return k(tokens, table, pe)


def kernel(tokens, table):
    pe = _pe_table()
    tperm = (tokens.reshape(BATCH, NW, NSTEP, KP)
             .transpose(1, 2, 0, 3).reshape(ROWS))
    out = _sc_embed(tperm, table, pe)
    return out.reshape(BATCH, MAX_LEN, D_MODEL)


# speedup vs baseline: 1.7171x; 1.7171x over previous
import functools

import jax
import jax.numpy as jnp
import numpy as np
from jax import lax
from jax.experimental import pallas as pl
from jax.experimental.pallas import tpu as pltpu
from jax.experimental.pallas import tpu_sc as plsc

D_MODEL = 2048
MAX_LEN = 2048
VOCAB = 80
BATCH = 4

ROWS = BATCH * MAX_LEN

_INFO = plsc.get_sparse_core_info()
NC, NS, L = _INFO.num_cores, _INFO.num_subcores, _INFO.num_lanes
NW = NC * NS
PPW = MAX_LEN // NW
KP = 8
NCH = PPW // KP
NSTEP = NCH * BATCH
NBUF = 3


def _pe_table_np():
    even_i = np.arange(0, D_MODEL, 2, dtype=np.float32)
    denominator = np.power(10000.0, even_i / D_MODEL)
    position = np.arange(MAX_LEN, dtype=np.float32).reshape(MAX_LEN, 1)
    even_pe = np.sin(position / denominator)
    odd_pe = np.cos(position / denominator)
    stacked = np.stack([even_pe, odd_pe], axis=2)
    return stacked.reshape(MAX_LEN, D_MODEL).astype(np.float32)


_PE = _pe_table_np()


def _sc_embed(tokens_flat, table, pe):
    mesh = plsc.VectorSubcoreMesh(core_axis_name="c", subcore_axis_name="s")

    @functools.partial(
        pl.kernel,
        mesh=mesh,
        out_type=jax.ShapeDtypeStruct((BATCH, MAX_LEN, D_MODEL), jnp.float32),
        scratch_types=[
            pltpu.VMEM((BATCH * PPW,), jnp.int32),
            pltpu.VMEM((KP, D_MODEL), jnp.float32),
            pltpu.VMEM((KP, D_MODEL), jnp.float32),
            pltpu.VMEM((KP, D_MODEL), jnp.float32),
            pltpu.VMEM((KP, D_MODEL), jnp.float32),
            pltpu.VMEM((KP, D_MODEL), jnp.float32),
            pltpu.SemaphoreType.DMA,
            pltpu.SemaphoreType.DMA,
            pltpu.SemaphoreType.DMA,
            pltpu.SemaphoreType.DMA,
            pltpu.SemaphoreType.DMA,
            pltpu.SemaphoreType.DMA,
            pltpu.SemaphoreType.DMA,
            pltpu.SemaphoreType.DMA,
        ],
    )
    def k(tok_hbm, table_hbm, pe_hbm, out_hbm, idx_v,
          g0, g1, g2, pb0, pb1,
          sg0, sg1, sg2, sp0, sp1, so0, so1, so2):
        wid = lax.axis_index("s") * NC + lax.axis_index("c")
        pos0 = wid * PPW

        G = [g0, g1, g2]
        P = [pb0, pb1]
        SG = [sg0, sg1, sg2]
        SP = [sp0, sp1]
        SO = [so0, so1, so2]

        for b in range(BATCH):
            pltpu.sync_copy(tok_hbm.at[pl.ds(b * MAX_LEN + pos0, PPW)],
                            idx_v.at[pl.ds(b * PPW, PPW)])

        def start_g(s):
            c, b = divmod(s, BATCH)
            kb = s % NBUF
            pltpu.async_copy(
                table_hbm.at[idx_v.at[pl.ds(b * PPW + c * KP, KP)]],
                G[kb], SG[kb])

        def wait_g(s):
            c, b = divmod(s, BATCH)
            kb = s % NBUF
            pltpu.make_async_copy(
                table_hbm.at[idx_v.at[pl.ds(b * PPW + c * KP, KP)]],
                G[kb], SG[kb]).wait()

        def start_pe(c):
            pltpu.async_copy(pe_hbm.at[pl.ds(pos0 + c * KP, KP)],
                             P[c % 2], SP[c % 2])

        def wait_pe(c):
            pltpu.make_async_copy(pe_hbm.at[pl.ds(pos0 + c * KP, KP)],
                                  P[c % 2], SP[c % 2]).wait()

        def add(s):
            kb = s % NBUF
            g, pbuf = G[kb], P[(s // BATCH) % 2]

            def body(i, acc):
                for r in range(KP):
                    g[r, pl.ds(i * L, L)] = (
                        g[r, pl.ds(i * L, L)] + pbuf[r, pl.ds(i * L, L)])
                return acc
            lax.fori_loop(0, D_MODEL // L, body, 0)

        def start_out(s):
            c, b = divmod(s, BATCH)
            kb = s % NBUF
            pltpu.async_copy(
                G[kb], out_hbm.at[b, pl.ds(pos0 + c * KP, KP)], SO[kb])

        def wait_out(s):
            c, b = divmod(s, BATCH)
            kb = s % NBUF
            pltpu.make_async_copy(
                G[kb], out_hbm.at[b, pl.ds(pos0 + c * KP, KP)], SO[kb]).wait()

        start_pe(0)
        start_pe(1)
        start_g(0)
        start_g(1)
        for s in range(NSTEP):
            c, b = divmod(s, BATCH)
            if b == 0:
                wait_pe(c)
            wait_g(s)
            add(s)
            start_out(s)
            if b == 3 and c + 2 < NCH:
                start_pe(c + 2)
            if s + 2 < NSTEP:
                if s >= 1:
                    wait_out(s - 1)
                start_g(s + 2)
        wait_out(NSTEP - 3)
        wait_out(NSTEP - 2)
        wait_out(NSTEP - 1)

    return k(tokens_flat, table, pe)


def kernel(tokens, table):
    return _sc_embed(tokens.reshape(ROWS), table, jnp.asarray(_PE))
